# Initial kernel scaffold; baseline (speedup 1.0000x reference)
import jax
import jax.numpy as jnp
from jax.experimental import pallas as pl


def kernel(x, edge_index, eps_noise, gl_weight, W1, b1, W2, b2, cW1, cb1, cW2, cb2):
    raise NotImplementedError("write your pallas kernel here")


# speedup vs baseline: 2.7380x; 2.7380x over previous
import functools

import jax
import jax.numpy as jnp
from jax import lax
from jax.experimental import pallas as pl
from jax.experimental.pallas import tpu as pltpu
from jax.experimental.pallas import tpu_sc as plsc

_G, _N, _D = 32, 256, 128
_P = 4
_HID = 256
_IB = 128
_NCLS = 10
_EPG = 4096
_EPSILON = 0.3
_SKIP = 0.2
_LANES = 16


def _sc_adjacency_body(ei, out, src_v, dst_v, adj_v):
    nc = 2
    wid = lax.axis_index("s") * nc + lax.axis_index("c")
    pltpu.sync_copy(ei.at[0, wid], src_v)
    pltpu.sync_copy(ei.at[1, wid], dst_v)

    zeros = jnp.zeros((_LANES,), jnp.float32)

    def zero_row(i, c):
        for j in range(_N // _LANES):
            adj_v[pl.ds(i * _N + j * _LANES, _LANES)] = zeros
        return c

    lax.fori_loop(0, _N, zero_row, 0)

    ones = jnp.ones((_LANES,), jnp.float32)

    def scatter_step(e, c):
        base = e * _LANES
        sv = src_v[pl.ds(base, _LANES)]
        dv = dst_v[pl.ds(base, _LANES)]
        plsc.store_scatter(adj_v, [sv * _N + dv], ones)
        return c

    lax.fori_loop(0, _EPG // _LANES, scatter_step, 0)
    pltpu.sync_copy(adj_v, out.at[wid])


@functools.partial(
    pl.kernel,
    mesh=plsc.VectorSubcoreMesh(core_axis_name="c", subcore_axis_name="s"),
    out_type=jax.ShapeDtypeStruct((_G, _N * _N), jnp.float32),
    compiler_params=pltpu.CompilerParams(needs_layout_passes=False),
    scratch_types=[
        pltpu.VMEM((_EPG,), jnp.int32),
        pltpu.VMEM((_EPG,), jnp.int32),
        pltpu.VMEM((_N * _N,), jnp.float32),
    ],
)
def _sc_adjacency(ei, out, src_v, dst_v, adj_v):
    _sc_adjacency_body(ei, out, src_v, dst_v, adj_v)


def _graph_body(xg_ref, raw_ref, glw_ref, w1_ref, b1_ref, w2_ref, b2_ref,
                out_ref):
    hi = lax.Precision.HIGHEST
    xg = xg_ref[...]
    raw = raw_ref[0]

    parts = []
    for p in range(_P):
        ex = xg * glw_ref[p:p + 1, :]
        s = jnp.sum(ex * ex, axis=1, keepdims=True)
        parts.append(ex / (jnp.sqrt(s) + 1e-12))
    a_feat = jnp.concatenate(parts, axis=1)
    att = lax.dot_general(a_feat, a_feat, (((1,), (1,)), ((), ())),
                          precision=hi,
                          preferred_element_type=jnp.float32) * (1.0 / _P)

    row = lax.broadcasted_iota(jnp.int32, (_N, _N), 0)
    col = lax.broadcasted_iota(jnp.int32, (_N, _N), 1)
    eye = jnp.where(row == col, 1.0, 0.0)
    bin_adj = jnp.where((raw > 0.0) | (att > _EPSILON), 1.0, 0.0)
    a = bin_adj + eye
    deg = jnp.sum(a, axis=1, keepdims=True)
    dinv = jnp.where(deg > 0.0, 1.0 / jnp.sqrt(deg), 0.0)

    xw1 = jnp.dot(xg, w1_ref[...], precision=hi)
    t1 = jnp.dot(a, dinv * xw1, precision=hi)
    h = jnp.maximum(dinv * t1 + b1_ref[...], 0.0)
    hw2 = jnp.dot(h, w2_ref[...], precision=hi)
    o = dinv * jnp.dot(a, dinv * hw2, precision=hi) + b2_ref[...]
    out_ref[0] = jnp.mean(o, axis=0, keepdims=True)


def _graph_stage(x, raw, glw_pad, w1, b1_2d, w2, b2_2d):
    return pl.pallas_call(
        _graph_body,
        grid=(_G,),
        in_specs=[
            pl.BlockSpec((_N, _D), lambda g: (g, 0)),
            pl.BlockSpec((1, _N, _N), lambda g: (g, 0, 0)),
            pl.BlockSpec((8, _D), lambda g: (0, 0)),
            pl.BlockSpec((_D, _HID), lambda g: (0, 0)),
            pl.BlockSpec((1, _HID), lambda g: (0, 0)),
            pl.BlockSpec((_HID, 2 * _IB), lambda g: (0, 0)),
            pl.BlockSpec((1, 2 * _IB), lambda g: (0, 0)),
        ],
        out_specs=pl.BlockSpec((1, 1, 2 * _IB), lambda g: (g, 0, 0)),
        out_shape=jax.ShapeDtypeStruct((_G, 1, 2 * _IB), jnp.float32),
    )(x, raw, glw_pad, w1, b1_2d, w2, b2_2d)


def _head_body(embs_ref, eps_ref, cw1_ref, cb1_ref, cw2_ref, cb2_ref,
               mu_ref, std_ref, lg_ref):
    hi = lax.Precision.HIGHEST
    embs = embs_ref[...]
    mu = embs[:, :_IB]
    t = embs[:, _IB:] - float(_IB)
    std = jnp.maximum(t, 0.0) + jnp.log1p(jnp.exp(-jnp.abs(t)))
    z = mu + eps_ref[...] * std
    hc = jnp.maximum(jnp.dot(z, cw1_ref[...], precision=hi) + cb1_ref[...],
                     0.0)
    lg = jnp.dot(hc, cw2_ref[...], precision=hi) + cb2_ref[...]
    mu_ref[...] = mu
    std_ref[...] = std
    lg_ref[...] = lg


def _head_stage(embs, eps, cw1, cb1_2d, cw2_pad, cb2_pad):
    return pl.pallas_call(
        _head_body,
        out_shape=(
            jax.ShapeDtypeStruct((_G, _IB), jnp.float32),
            jax.ShapeDtypeStruct((_G, _IB), jnp.float32),
            jax.ShapeDtypeStruct((_G, _IB), jnp.float32),
        ),
    )(embs, eps, cw1, cb1_2d, cw2_pad, cb2_pad)


def kernel(x, edge_index, eps_noise, gl_weight, W1, b1, W2, b2,
           cW1, cb1, cW2, cb2):
    ei = edge_index.astype(jnp.int32)
    raw = _sc_adjacency(ei).reshape(_G, _N, _N)

    glw_pad = jnp.pad(gl_weight, ((0, 8 - _P), (0, 0)))
    embs = _graph_stage(x, raw, glw_pad, W1, b1.reshape(1, -1), W2,
                        b2.reshape(1, -1)).reshape(_G, 2 * _IB)

    cw2_pad = jnp.pad(cW2, ((0, 0), (0, _IB - _NCLS)))
    cb2_pad = jnp.pad(cb2, (0, _IB - _NCLS)).reshape(1, -1)
    mu, std, lg = _head_stage(embs, eps_noise, cW1, cb1.reshape(1, -1),
                              cw2_pad, cb2_pad)
    return (mu, std, lg[:, :_NCLS])


# speedup vs baseline: 4.2174x; 1.5403x over previous
import functools

import jax
import jax.numpy as jnp
from jax import lax
from jax.experimental import pallas as pl
from jax.experimental.pallas import tpu as pltpu
from jax.experimental.pallas import tpu_sc as plsc

_G, _N, _D = 32, 256, 128
_P = 4
_HID = 256
_IB = 128
_NCLS = 10
_EPG = 4096
_EPSILON = 0.3
_SKIP = 0.2
_LANES = 16


def _sc_adjacency_body(ei, out, src_v, dst_v, adj_v):
    nc = 2
    wid = lax.axis_index("s") * nc + lax.axis_index("c")
    pltpu.sync_copy(ei.at[0, wid], src_v)
    pltpu.sync_copy(ei.at[1, wid], dst_v)

    zeros = jnp.zeros((_LANES,), jnp.float32)

    def zero_row(i, c):
        for j in range(_N // _LANES):
            adj_v[i, pl.ds(j * _LANES, _LANES)] = zeros
        return c

    lax.fori_loop(0, _N, zero_row, 0)

    ones = jnp.ones((_LANES,), jnp.float32)

    def scatter_step(e, c):
        base = e * _LANES
        sv = src_v[pl.ds(base, _LANES)]
        dv = dst_v[pl.ds(base, _LANES)]
        plsc.store_scatter(adj_v, [sv, dv], ones)
        return c

    lax.fori_loop(0, _EPG // _LANES, scatter_step, 0)
    pltpu.sync_copy(adj_v, out.at[wid])


@functools.partial(
    pl.kernel,
    mesh=plsc.VectorSubcoreMesh(core_axis_name="c", subcore_axis_name="s"),
    out_type=jax.ShapeDtypeStruct((_G, _N, _N), jnp.float32),
    compiler_params=pltpu.CompilerParams(needs_layout_passes=False),
    scratch_types=[
        pltpu.VMEM((_EPG,), jnp.int32),
        pltpu.VMEM((_EPG,), jnp.int32),
        pltpu.VMEM((_N, _N), jnp.float32),
    ],
)
def _sc_adjacency(ei, out, src_v, dst_v, adj_v):
    _sc_adjacency_body(ei, out, src_v, dst_v, adj_v)


def _graph_body(xg_ref, raw_ref, glw_ref, w1_ref, b1_ref, w2_ref, b2_ref,
                out_ref):
    hi = lax.Precision.HIGHEST
    xg = xg_ref[...]
    raw = raw_ref[0]

    parts = []
    for p in range(_P):
        ex = xg * glw_ref[p:p + 1, :]
        s = jnp.sum(ex * ex, axis=1, keepdims=True)
        parts.append(ex / (jnp.sqrt(s) + 1e-12))
    a_feat = jnp.concatenate(parts, axis=1)
    att = lax.dot_general(a_feat, a_feat, (((1,), (1,)), ((), ())),
                          precision=hi,
                          preferred_element_type=jnp.float32) * (1.0 / _P)

    row = lax.broadcasted_iota(jnp.int32, (_N, _N), 0)
    col = lax.broadcasted_iota(jnp.int32, (_N, _N), 1)
    eye = jnp.where(row == col, 1.0, 0.0)
    bin_adj = jnp.where((raw > 0.0) | (att > _EPSILON), 1.0, 0.0)
    a = bin_adj + eye
    deg = jnp.sum(a, axis=1, keepdims=True)
    dinv = jnp.where(deg > 0.0, 1.0 / jnp.sqrt(deg), 0.0)

    lo = lax.Precision.DEFAULT
    xw1 = jnp.dot(xg, w1_ref[...], precision=lo)
    t1 = jnp.dot(a, dinv * xw1, precision=lo)
    h = jnp.maximum(dinv * t1 + b1_ref[...], 0.0)
    hw2 = jnp.dot(h, w2_ref[...], precision=lo)
    o = dinv * jnp.dot(a, dinv * hw2, precision=lo) + b2_ref[...]
    out_ref[0] = jnp.mean(o, axis=0, keepdims=True)


def _graph_stage(x, raw, glw_pad, w1, b1_2d, w2, b2_2d):
    return pl.pallas_call(
        _graph_body,
        grid=(_G,),
        in_specs=[
            pl.BlockSpec((_N, _D), lambda g: (g, 0)),
            pl.BlockSpec((1, _N, _N), lambda g: (g, 0, 0)),
            pl.BlockSpec((8, _D), lambda g: (0, 0)),
            pl.BlockSpec((_D, _HID), lambda g: (0, 0)),
            pl.BlockSpec((1, _HID), lambda g: (0, 0)),
            pl.BlockSpec((_HID, 2 * _IB), lambda g: (0, 0)),
            pl.BlockSpec((1, 2 * _IB), lambda g: (0, 0)),
        ],
        out_specs=pl.BlockSpec((1, 1, 2 * _IB), lambda g: (g, 0, 0)),
        out_shape=jax.ShapeDtypeStruct((_G, 1, 2 * _IB), jnp.float32),
    )(x, raw, glw_pad, w1, b1_2d, w2, b2_2d)


def _head_body(embs_ref, eps_ref, cw1_ref, cb1_ref, cw2_ref, cb2_ref,
               mu_ref, std_ref, lg_ref):
    hi = lax.Precision.HIGHEST
    embs = embs_ref[...]
    mu = embs[:, :_IB]
    t = embs[:, _IB:] - float(_IB)
    std = jnp.maximum(t, 0.0) + jnp.log1p(jnp.exp(-jnp.abs(t)))
    z = mu + eps_ref[...] * std
    hc = jnp.maximum(jnp.dot(z, cw1_ref[...], precision=hi) + cb1_ref[...],
                     0.0)
    lg = jnp.dot(hc, cw2_ref[...], precision=hi) + cb2_ref[...]
    mu_ref[...] = mu
    std_ref[...] = std
    lg_ref[...] = lg


def _head_stage(embs, eps, cw1, cb1_2d, cw2_pad, cb2_pad):
    return pl.pallas_call(
        _head_body,
        out_shape=(
            jax.ShapeDtypeStruct((_G, _IB), jnp.float32),
            jax.ShapeDtypeStruct((_G, _IB), jnp.float32),
            jax.ShapeDtypeStruct((_G, _IB), jnp.float32),
        ),
    )(embs, eps, cw1, cb1_2d, cw2_pad, cb2_pad)


def kernel(x, edge_index, eps_noise, gl_weight, W1, b1, W2, b2,
           cW1, cb1, cW2, cb2):
    ei = edge_index.astype(jnp.int32)
    raw = _sc_adjacency(ei)

    glw_pad = jnp.pad(gl_weight, ((0, 8 - _P), (0, 0)))
    embs = _graph_stage(x, raw, glw_pad, W1, b1.reshape(1, -1), W2,
                        b2.reshape(1, -1)).reshape(_G, 2 * _IB)

    cw2_pad = jnp.pad(cW2, ((0, 0), (0, _IB - _NCLS)))
    cb2_pad = jnp.pad(cb2, (0, _IB - _NCLS)).reshape(1, -1)
    mu, std, lg = _head_stage(embs, eps_noise, cW1, cb1.reshape(1, -1),
                              cw2_pad, cb2_pad)
    return (mu, std, lg[:, :_NCLS])


# speedup vs baseline: 5.2213x; 1.2381x over previous
import functools

import jax
import jax.numpy as jnp
from jax import lax
from jax.experimental import pallas as pl
from jax.experimental.pallas import tpu as pltpu
from jax.experimental.pallas import tpu_sc as plsc

_G, _N, _D = 32, 256, 128
_P = 4
_HID = 256
_IB = 128
_NCLS = 10
_EPG = 4096
_EPSILON = 0.3
_SKIP = 0.2
_LANES = 16


def _sc_adjacency_body(ei, out, src_v, dst_v, adj_v):
    nc = 2
    wid = lax.axis_index("s") * nc + lax.axis_index("c")
    pltpu.sync_copy(ei.at[0, wid], src_v)
    pltpu.sync_copy(ei.at[1, wid], dst_v)

    zeros = jnp.zeros((_LANES,), jnp.float32)

    def zero_row(i, c):
        for j in range(_N // _LANES):
            adj_v[i, pl.ds(j * _LANES, _LANES)] = zeros
        return c

    lax.fori_loop(0, _N, zero_row, 0)

    ones = jnp.ones((_LANES,), jnp.float32)

    def scatter_step(e, c):
        base = e * _LANES
        sv = src_v[pl.ds(base, _LANES)]
        dv = dst_v[pl.ds(base, _LANES)]
        plsc.store_scatter(adj_v, [sv, dv], ones)
        return c

    lax.fori_loop(0, _EPG // _LANES, scatter_step, 0)
    pltpu.sync_copy(adj_v, out.at[wid])


@functools.partial(
    pl.kernel,
    mesh=plsc.VectorSubcoreMesh(core_axis_name="c", subcore_axis_name="s"),
    out_type=jax.ShapeDtypeStruct((_G, _N, _N), jnp.float32),
    compiler_params=pltpu.CompilerParams(needs_layout_passes=False),
    scratch_types=[
        pltpu.VMEM((_EPG,), jnp.int32),
        pltpu.VMEM((_EPG,), jnp.int32),
        pltpu.VMEM((_N, _N), jnp.float32),
    ],
)
def _sc_adjacency(ei, out, src_v, dst_v, adj_v):
    _sc_adjacency_body(ei, out, src_v, dst_v, adj_v)


def _graph_body(xg_ref, raw_ref, glw_ref, w1_ref, b1_ref, w2_ref, b2_ref,
                out_ref):
    hi = lax.Precision.DEFAULT
    xg = xg_ref[...]
    raw = raw_ref[0]

    parts = []
    for p in range(_P):
        ex = xg * glw_ref[p:p + 1, :]
        s = jnp.sum(ex * ex, axis=1, keepdims=True)
        parts.append(ex / (jnp.sqrt(s) + 1e-12))
    a_feat = jnp.concatenate(parts, axis=1)
    att = lax.dot_general(a_feat, a_feat, (((1,), (1,)), ((), ())),
                          precision=hi,
                          preferred_element_type=jnp.float32) * (1.0 / _P)

    row = lax.broadcasted_iota(jnp.int32, (_N, _N), 0)
    col = lax.broadcasted_iota(jnp.int32, (_N, _N), 1)
    eye = jnp.where(row == col, 1.0, 0.0)
    bin_adj = jnp.where((raw > 0.0) | (att > _EPSILON), 1.0, 0.0)
    a = bin_adj + eye
    deg = jnp.sum(a, axis=1, keepdims=True)
    dinv = jnp.where(deg > 0.0, 1.0 / jnp.sqrt(deg), 0.0)

    lo = lax.Precision.DEFAULT
    xw1 = jnp.dot(xg, w1_ref[...], precision=lo)
    t1 = jnp.dot(a, dinv * xw1, precision=lo)
    h = jnp.maximum(dinv * t1 + b1_ref[...], 0.0)
    hw2 = jnp.dot(h, w2_ref[...], precision=lo)
    o = dinv * jnp.dot(a, dinv * hw2, precision=lo) + b2_ref[...]
    out_ref[0] = jnp.mean(o, axis=0, keepdims=True)


def _graph_stage(x, raw, glw_pad, w1, b1_2d, w2, b2_2d):
    return pl.pallas_call(
        _graph_body,
        grid=(_G,),
        in_specs=[
            pl.BlockSpec((_N, _D), lambda g: (g, 0)),
            pl.BlockSpec((1, _N, _N), lambda g: (g, 0, 0)),
            pl.BlockSpec((8, _D), lambda g: (0, 0)),
            pl.BlockSpec((_D, _HID), lambda g: (0, 0)),
            pl.BlockSpec((1, _HID), lambda g: (0, 0)),
            pl.BlockSpec((_HID, 2 * _IB), lambda g: (0, 0)),
            pl.BlockSpec((1, 2 * _IB), lambda g: (0, 0)),
        ],
        out_specs=pl.BlockSpec((1, 1, 2 * _IB), lambda g: (g, 0, 0)),
        out_shape=jax.ShapeDtypeStruct((_G, 1, 2 * _IB), jnp.float32),
    )(x, raw, glw_pad, w1, b1_2d, w2, b2_2d)


def _head_body(embs_ref, eps_ref, cw1_ref, cb1_ref, cw2_ref, cb2_ref,
               mu_ref, std_ref, lg_ref):
    hi = lax.Precision.HIGHEST
    embs = embs_ref[...]
    mu = embs[:, :_IB]
    t = embs[:, _IB:] - float(_IB)
    std = jnp.maximum(t, 0.0) + jnp.log1p(jnp.exp(-jnp.abs(t)))
    z = mu + eps_ref[...] * std
    hc = jnp.maximum(jnp.dot(z, cw1_ref[...], precision=hi) + cb1_ref[...],
                     0.0)
    lg = jnp.dot(hc, cw2_ref[...], precision=hi) + cb2_ref[...]
    mu_ref[...] = mu
    std_ref[...] = std
    lg_ref[...] = lg


def _head_stage(embs, eps, cw1, cb1_2d, cw2_pad, cb2_pad):
    return pl.pallas_call(
        _head_body,
        out_shape=(
            jax.ShapeDtypeStruct((_G, _IB), jnp.float32),
            jax.ShapeDtypeStruct((_G, _IB), jnp.float32),
            jax.ShapeDtypeStruct((_G, _IB), jnp.float32),
        ),
    )(embs, eps, cw1, cb1_2d, cw2_pad, cb2_pad)


def kernel(x, edge_index, eps_noise, gl_weight, W1, b1, W2, b2,
           cW1, cb1, cW2, cb2):
    ei = edge_index.astype(jnp.int32)
    raw = _sc_adjacency(ei)

    glw_pad = jnp.pad(gl_weight, ((0, 8 - _P), (0, 0)))
    embs = _graph_stage(x, raw, glw_pad, W1, b1.reshape(1, -1), W2,
                        b2.reshape(1, -1)).reshape(_G, 2 * _IB)

    cw2_pad = jnp.pad(cW2, ((0, 0), (0, _IB - _NCLS)))
    cb2_pad = jnp.pad(cb2, (0, _IB - _NCLS)).reshape(1, -1)
    mu, std, lg = _head_stage(embs, eps_noise, cW1, cb1.reshape(1, -1),
                              cw2_pad, cb2_pad)
    return (mu, std, lg[:, :_NCLS])


# speedup vs baseline: 5.3098x; 1.0169x over previous
import functools

import jax
import jax.numpy as jnp
import numpy as np
from jax import lax
from jax.experimental import pallas as pl
from jax.experimental.pallas import tpu as pltpu
from jax.experimental.pallas import tpu_sc as plsc

_G, _N, _D = 32, 256, 128
_P = 4
_HID = 256
_IB = 128
_NCLS = 10
_EPG = 4096
_EPSILON = 0.3
_SKIP = 0.2
_LANES = 16


def _sc_adjacency_body(ei, out, src_v, dst_v, adj_v):
    nc = 2
    wid = lax.axis_index("s") * nc + lax.axis_index("c")
    pltpu.sync_copy(ei.at[0, wid], src_v)
    pltpu.sync_copy(ei.at[1, wid], dst_v)

    zeros = jnp.zeros((_LANES,), jnp.float32)

    def zero_row(i, c):
        for j in range(_N // _LANES):
            adj_v[i, pl.ds(j * _LANES, _LANES)] = zeros
        return c

    lax.fori_loop(0, _N, zero_row, 0)

    ones = jnp.ones((_LANES,), jnp.float32)

    def scatter_step(e, c):
        base = e * _LANES
        sv = src_v[pl.ds(base, _LANES)]
        dv = dst_v[pl.ds(base, _LANES)]
        plsc.store_scatter(adj_v, [sv, dv], ones)
        return c

    lax.fori_loop(0, _EPG // _LANES, scatter_step, 0)
    pltpu.sync_copy(adj_v, out.at[wid])


@functools.partial(
    pl.kernel,
    mesh=plsc.VectorSubcoreMesh(core_axis_name="c", subcore_axis_name="s"),
    out_type=jax.ShapeDtypeStruct((_G, _N, _N), jnp.float32),
    compiler_params=pltpu.CompilerParams(needs_layout_passes=False),
    scratch_types=[
        pltpu.VMEM((_EPG,), jnp.int32),
        pltpu.VMEM((_EPG,), jnp.int32),
        pltpu.VMEM((_N, _N), jnp.float32),
    ],
)
def _sc_adjacency(ei, out, src_v, dst_v, adj_v):
    _sc_adjacency_body(ei, out, src_v, dst_v, adj_v)


def _graph_body(xg_ref, raw_ref, glw_ref, w1_ref, b1_ref, w2_ref,
                b2_ref, out_ref):
    lo = lax.Precision.DEFAULT
    xg = xg_ref[...]
    raw = raw_ref[0]

    parts = []
    for p in range(_P):
        ex = xg * glw_ref[p:p + 1, :]
        s = jnp.sum(ex * ex, axis=1, keepdims=True)
        parts.append(ex * lax.rsqrt(s))
    a_feat = jnp.concatenate(parts, axis=1)
    att = lax.dot_general(a_feat, a_feat, (((1,), (1,)), ((), ())),
                          precision=lo,
                          preferred_element_type=jnp.float32)

    thresh = _P * float(np.float32(_EPSILON))
    bin_adj = jnp.where((raw > 0.0) | (att > thresh), 1.0, 0.0)
    deg = jnp.sum(bin_adj, axis=1, keepdims=True) + 1.0
    dinv = lax.rsqrt(deg)

    xw1 = jnp.dot(xg, w1_ref[...], precision=lo)
    dxw1 = dinv * xw1
    t1 = jnp.dot(bin_adj, dxw1, precision=lo) + dxw1
    h = jnp.maximum(dinv * t1 + b1_ref[...], 0.0)
    hw2 = jnp.dot(h, w2_ref[...], precision=lo)
    dhw2 = dinv * hw2
    o = dinv * (jnp.dot(bin_adj, dhw2, precision=lo) + dhw2) + b2_ref[...]
    out_ref[0] = jnp.mean(o, axis=0, keepdims=True)


def _graph_stage(x, raw, glw_pad, w1, b1_2d, w2, b2_2d):
    return pl.pallas_call(
        _graph_body,
        grid=(_G,),
        in_specs=[
            pl.BlockSpec((_N, _D), lambda g: (g, 0)),
            pl.BlockSpec((1, _N, _N), lambda g: (g, 0, 0)),
            pl.BlockSpec((8, _D), lambda g: (0, 0)),
            pl.BlockSpec((_D, _HID), lambda g: (0, 0)),
            pl.BlockSpec((1, _HID), lambda g: (0, 0)),
            pl.BlockSpec((_HID, 2 * _IB), lambda g: (0, 0)),
            pl.BlockSpec((1, 2 * _IB), lambda g: (0, 0)),
        ],
        out_specs=pl.BlockSpec((1, 1, 2 * _IB), lambda g: (g, 0, 0)),
        out_shape=jax.ShapeDtypeStruct((_G, 1, 2 * _IB), jnp.float32),
    )(x, raw, glw_pad, w1, b1_2d, w2, b2_2d)


def _head_body(embs_ref, eps_ref, cw1_ref, cb1_ref, cw2_ref, cb2_ref,
               mu_ref, std_ref, lg_ref):
    hi = lax.Precision.HIGHEST
    embs = embs_ref[...]
    mu = embs[:, :_IB]
    t = embs[:, _IB:] - float(_IB)
    std = jnp.maximum(t, 0.0) + jnp.log1p(jnp.exp(-jnp.abs(t)))
    z = mu + eps_ref[...] * std
    hc = jnp.maximum(jnp.dot(z, cw1_ref[...], precision=hi) + cb1_ref[...],
                     0.0)
    lg = jnp.dot(hc, cw2_ref[...], precision=hi) + cb2_ref[...]
    mu_ref[...] = mu
    std_ref[...] = std
    lg_ref[...] = lg


def _head_stage(embs, eps, cw1, cb1_2d, cw2_pad, cb2_pad):
    return pl.pallas_call(
        _head_body,
        out_shape=(
            jax.ShapeDtypeStruct((_G, _IB), jnp.float32),
            jax.ShapeDtypeStruct((_G, _IB), jnp.float32),
            jax.ShapeDtypeStruct((_G, _IB), jnp.float32),
        ),
    )(embs, eps, cw1, cb1_2d, cw2_pad, cb2_pad)


def kernel(x, edge_index, eps_noise, gl_weight, W1, b1, W2, b2,
           cW1, cb1, cW2, cb2):
    ei = edge_index.astype(jnp.int32)
    raw = _sc_adjacency(ei)

    glw_pad = jnp.pad(gl_weight, ((0, 8 - _P), (0, 0)))
    embs = _graph_stage(x, raw, glw_pad, W1, b1.reshape(1, -1), W2,
                        b2.reshape(1, -1)).reshape(_G, 2 * _IB)

    cw2_pad = jnp.pad(cW2, ((0, 0), (0, _IB - _NCLS)))
    cb2_pad = jnp.pad(cb2, (0, _IB - _NCLS)).reshape(1, -1)
    mu, std, lg = _head_stage(embs, eps_noise, cW1, cb1.reshape(1, -1),
                              cw2_pad, cb2_pad)
    return (mu, std, lg[:, :_NCLS])


# speedup vs baseline: 5.6424x; 1.0626x over previous
import functools

import jax
import jax.numpy as jnp
import numpy as np
from jax import lax
from jax.experimental import pallas as pl
from jax.experimental.pallas import tpu as pltpu
from jax.experimental.pallas import tpu_sc as plsc

_G, _N, _D = 32, 256, 128
_P = 4
_HID = 256
_IB = 128
_NCLS = 10
_EPG = 4096
_EPSILON = 0.3
_SKIP = 0.2
_LANES = 16


def _sc_adjacency_body(ei, out, src_v, dst_v, adj_v, sem):
    nc = 2
    wid = lax.axis_index("s") * nc + lax.axis_index("c")
    cp_src = pltpu.async_copy(ei.at[0, wid], src_v, sem)
    cp_dst = pltpu.async_copy(ei.at[1, wid], dst_v, sem)

    zeros = jnp.zeros((_LANES,), jnp.float32)

    def zero_row(i, c):
        for j in range(_N // _LANES):
            adj_v[i, pl.ds(j * _LANES, _LANES)] = zeros
        return c

    lax.fori_loop(0, _N, zero_row, 0)
    cp_src.wait()
    cp_dst.wait()

    ones = jnp.ones((_LANES,), jnp.float32)

    def scatter_step(e, c):
        base = e * _LANES
        sv = src_v[pl.ds(base, _LANES)]
        dv = dst_v[pl.ds(base, _LANES)]
        plsc.store_scatter(adj_v, [sv, dv], ones)
        return c

    lax.fori_loop(0, _EPG // _LANES, scatter_step, 0)
    pltpu.sync_copy(adj_v, out.at[wid])


@functools.partial(
    pl.kernel,
    mesh=plsc.VectorSubcoreMesh(core_axis_name="c", subcore_axis_name="s"),
    out_type=jax.ShapeDtypeStruct((_G, _N, _N), jnp.float32),
    compiler_params=pltpu.CompilerParams(needs_layout_passes=False),
    scratch_types=[
        pltpu.VMEM((_EPG,), jnp.int32),
        pltpu.VMEM((_EPG,), jnp.int32),
        pltpu.VMEM((_N, _N), jnp.float32),
        pltpu.SemaphoreType.DMA,
    ],
)
def _sc_adjacency(ei, out, src_v, dst_v, adj_v, sem):
    _sc_adjacency_body(ei, out, src_v, dst_v, adj_v, sem)


def _graph_body(xg_ref, raw_ref, glw_ref, w1_ref, b1_ref, w2_ref, b2_ref,
                eps_ref, cw1_ref, cb1_ref, cw2_ref, cb2_ref,
                mu_ref, std_ref, lg_ref, embs_scr):
    lo = lax.Precision.DEFAULT
    g = pl.program_id(0)
    xg = xg_ref[...]
    raw = raw_ref[0]

    parts = []
    for p in range(_P):
        ex = xg * glw_ref[p:p + 1, :]
        s = jnp.sum(ex * ex, axis=1, keepdims=True)
        parts.append(ex * lax.rsqrt(s))
    a_feat = jnp.concatenate(parts, axis=1)
    att = lax.dot_general(a_feat, a_feat, (((1,), (1,)), ((), ())),
                          precision=lo,
                          preferred_element_type=jnp.float32)

    thresh = _P * float(np.float32(_EPSILON))
    bin_adj = jnp.where((raw > 0.0) | (att > thresh), 1.0, 0.0)
    deg = jnp.sum(bin_adj, axis=1, keepdims=True) + 1.0
    dinv = lax.rsqrt(deg)

    xw1 = jnp.dot(xg, w1_ref[...], precision=lo)
    dxw1 = dinv * xw1
    t1 = jnp.dot(bin_adj, dxw1, precision=lo) + dxw1
    h = jnp.maximum(dinv * t1 + b1_ref[...], 0.0)
    hw2 = jnp.dot(h, w2_ref[...], precision=lo)
    dhw2 = dinv * hw2
    o = dinv * (jnp.dot(bin_adj, dhw2, precision=lo) + dhw2) + b2_ref[...]
    embs_scr[pl.ds(g, 1), :] = jnp.mean(o, axis=0, keepdims=True)

    @pl.when(g == _G - 1)
    def _head():
        embs = embs_scr[...]
        mu = embs[:, :_IB]
        t = embs[:, _IB:] - float(_IB)
        std = jnp.maximum(t, 0.0) + jnp.log1p(jnp.exp(-jnp.abs(t)))
        z = mu + eps_ref[...] * std
        hc = jnp.maximum(
            jnp.dot(z, cw1_ref[...], precision=lo) + cb1_ref[...], 0.0)
        lg_ref[...] = jnp.dot(hc, cw2_ref[...], precision=lo) + cb2_ref[...]
        mu_ref[...] = mu
        std_ref[...] = std


def _graph_stage(x, raw, glw_pad, w1, b1_2d, w2, b2_2d,
                 eps, cw1, cb1_2d, cw2_pad, cb2_pad):
    zero2 = lambda g: (0, 0)
    return pl.pallas_call(
        _graph_body,
        grid=(_G,),
        in_specs=[
            pl.BlockSpec((_N, _D), lambda g: (g, 0)),
            pl.BlockSpec((1, _N, _N), lambda g: (g, 0, 0)),
            pl.BlockSpec((8, _D), zero2),
            pl.BlockSpec((_D, _HID), zero2),
            pl.BlockSpec((1, _HID), zero2),
            pl.BlockSpec((_HID, 2 * _IB), zero2),
            pl.BlockSpec((1, 2 * _IB), zero2),
            pl.BlockSpec((_G, _IB), zero2),
            pl.BlockSpec((_IB, _IB), zero2),
            pl.BlockSpec((1, _IB), zero2),
            pl.BlockSpec((_IB, _IB), zero2),
            pl.BlockSpec((1, _IB), zero2),
        ],
        out_specs=(
            pl.BlockSpec((_G, _IB), zero2),
            pl.BlockSpec((_G, _IB), zero2),
            pl.BlockSpec((_G, _IB), zero2),
        ),
        out_shape=(
            jax.ShapeDtypeStruct((_G, _IB), jnp.float32),
            jax.ShapeDtypeStruct((_G, _IB), jnp.float32),
            jax.ShapeDtypeStruct((_G, _IB), jnp.float32),
        ),
        scratch_shapes=[pltpu.VMEM((_G, 2 * _IB), jnp.float32)],
    )(x, raw, glw_pad, w1, b1_2d, w2, b2_2d, eps, cw1, cb1_2d, cw2_pad,
      cb2_pad)


def kernel(x, edge_index, eps_noise, gl_weight, W1, b1, W2, b2,
           cW1, cb1, cW2, cb2):
    ei = edge_index.astype(jnp.int32)
    raw = _sc_adjacency(ei)

    glw_pad = jnp.pad(gl_weight, ((0, 8 - _P), (0, 0)))
    cw2_pad = jnp.pad(cW2, ((0, 0), (0, _IB - _NCLS)))
    cb2_pad = jnp.pad(cb2, (0, _IB - _NCLS)).reshape(1, -1)
    mu, std, lg = _graph_stage(x, raw, glw_pad, W1, b1.reshape(1, -1), W2,
                               b2.reshape(1, -1), eps_noise, cW1,
                               cb1.reshape(1, -1), cw2_pad, cb2_pad)
    return (mu, std, lg[:, :_NCLS])


# speedup vs baseline: 6.4305x; 1.1397x over previous
import functools

import jax
import jax.numpy as jnp
import numpy as np
from jax import lax
from jax.experimental import pallas as pl
from jax.experimental.pallas import tpu as pltpu
from jax.experimental.pallas import tpu_sc as plsc

_G, _N, _D = 32, 256, 128
_P = 4
_HID = 256
_IB = 128
_NCLS = 10
_EPG = 4096
_EPSILON = 0.3
_SKIP = 0.2
_LANES = 16


def _sc_adjacency_body(ei, out, src_v, dst_v, adj_v, sem):
    nc = 2
    wid = lax.axis_index("s") * nc + lax.axis_index("c")
    cp_src = pltpu.async_copy(ei.at[0, wid], src_v, sem)
    cp_dst = pltpu.async_copy(ei.at[1, wid], dst_v, sem)

    zeros = jnp.zeros((_LANES,), jnp.float32)

    def zero_row(i, c):
        for j in range(_N // _LANES):
            adj_v[i, pl.ds(j * _LANES, _LANES)] = zeros
        return c

    lax.fori_loop(0, _N, zero_row, 0)
    cp_src.wait()
    cp_dst.wait()

    ones = jnp.ones((_LANES,), jnp.float32)

    def scatter_step(e, c):
        base = e * _LANES
        sv = src_v[pl.ds(base, _LANES)]
        dv = dst_v[pl.ds(base, _LANES)]
        plsc.store_scatter(adj_v, [sv, dv], ones)
        return c

    lax.fori_loop(0, _EPG // _LANES, scatter_step, 0)
    pltpu.sync_copy(adj_v, out.at[wid])


@functools.partial(
    pl.kernel,
    mesh=plsc.VectorSubcoreMesh(core_axis_name="c", subcore_axis_name="s"),
    out_type=jax.ShapeDtypeStruct((_G, _N, _N), jnp.float32),
    compiler_params=pltpu.CompilerParams(needs_layout_passes=False),
    scratch_types=[
        pltpu.VMEM((_EPG,), jnp.int32),
        pltpu.VMEM((_EPG,), jnp.int32),
        pltpu.VMEM((_N, _N), jnp.float32),
        pltpu.SemaphoreType.DMA,
    ],
)
def _sc_adjacency(ei, out, src_v, dst_v, adj_v, sem):
    _sc_adjacency_body(ei, out, src_v, dst_v, adj_v, sem)


_GPB = 2


def _graph_body(xg_ref, raw_ref, glw_ref, w1_ref, b1_ref, w2_ref, b2_ref,
                eps_ref, cw1_ref, cb1_ref, cw2_ref, cb2_ref,
                mu_ref, std_ref, lg_ref, embs_scr):
    lo = lax.Precision.DEFAULT
    g = pl.program_id(0)
    thresh = _P * float(np.float32(_EPSILON))

    for k in range(_GPB):
        xg = xg_ref[pl.ds(k * _N, _N), :]
        raw = raw_ref[k]

        parts = []
        for p in range(_P):
            ex = xg * glw_ref[p:p + 1, :]
            s = jnp.sum(ex * ex, axis=1, keepdims=True)
            parts.append(ex * lax.rsqrt(s))
        a_feat = jnp.concatenate(parts, axis=1)
        att = lax.dot_general(a_feat, a_feat, (((1,), (1,)), ((), ())),
                              precision=lo,
                              preferred_element_type=jnp.float32)

        bin_adj = jnp.where((raw > 0.0) | (att > thresh), 1.0, 0.0)
        deg = jnp.sum(bin_adj, axis=1, keepdims=True) + 1.0
        dinv = lax.rsqrt(deg)

        xw1 = jnp.dot(xg, w1_ref[...], precision=lo)
        dxw1 = dinv * xw1
        t1 = jnp.dot(bin_adj, dxw1, precision=lo) + dxw1
        h = jnp.maximum(dinv * t1 + b1_ref[...], 0.0)
        hw2 = jnp.dot(h, w2_ref[...], precision=lo)
        dhw2 = dinv * hw2
        o = (dinv * (jnp.dot(bin_adj, dhw2, precision=lo) + dhw2)
             + b2_ref[...])
        embs_scr[pl.ds(g * _GPB + k, 1), :] = jnp.mean(o, axis=0,
                                                       keepdims=True)

    @pl.when(g == _G // _GPB - 1)
    def _head():
        embs = embs_scr[...]
        mu = embs[:, :_IB]
        t = embs[:, _IB:] - float(_IB)
        std = jnp.maximum(t, 0.0) + jnp.log1p(jnp.exp(-jnp.abs(t)))
        z = mu + eps_ref[...] * std
        hc = jnp.maximum(
            jnp.dot(z, cw1_ref[...], precision=lo) + cb1_ref[...], 0.0)
        lg_ref[...] = jnp.dot(hc, cw2_ref[...], precision=lo) + cb2_ref[...]
        mu_ref[...] = mu
        std_ref[...] = std


def _graph_stage(x, raw, glw_pad, w1, b1_2d, w2, b2_2d,
                 eps, cw1, cb1_2d, cw2_pad, cb2_pad):
    zero2 = lambda g: (0, 0)
    return pl.pallas_call(
        _graph_body,
        grid=(_G // _GPB,),
        in_specs=[
            pl.BlockSpec((_GPB * _N, _D), lambda g: (g, 0)),
            pl.BlockSpec((_GPB, _N, _N), lambda g: (g, 0, 0)),
            pl.BlockSpec((8, _D), zero2),
            pl.BlockSpec((_D, _HID), zero2),
            pl.BlockSpec((1, _HID), zero2),
            pl.BlockSpec((_HID, 2 * _IB), zero2),
            pl.BlockSpec((1, 2 * _IB), zero2),
            pl.BlockSpec((_G, _IB), zero2),
            pl.BlockSpec((_IB, _IB), zero2),
            pl.BlockSpec((1, _IB), zero2),
            pl.BlockSpec((_IB, _IB), zero2),
            pl.BlockSpec((1, _IB), zero2),
        ],
        out_specs=(
            pl.BlockSpec((_G, _IB), zero2),
            pl.BlockSpec((_G, _IB), zero2),
            pl.BlockSpec((_G, _IB), zero2),
        ),
        out_shape=(
            jax.ShapeDtypeStruct((_G, _IB), jnp.float32),
            jax.ShapeDtypeStruct((_G, _IB), jnp.float32),
            jax.ShapeDtypeStruct((_G, _IB), jnp.float32),
        ),
        scratch_shapes=[pltpu.VMEM((_G, 2 * _IB), jnp.float32)],
    )(x, raw, glw_pad, w1, b1_2d, w2, b2_2d, eps, cw1, cb1_2d, cw2_pad,
      cb2_pad)


def kernel(x, edge_index, eps_noise, gl_weight, W1, b1, W2, b2,
           cW1, cb1, cW2, cb2):
    ei = edge_index.astype(jnp.int32)
    raw = _sc_adjacency(ei)

    glw_pad = jnp.pad(gl_weight, ((0, 8 - _P), (0, 0)))
    cw2_pad = jnp.pad(cW2, ((0, 0), (0, _IB - _NCLS)))
    cb2_pad = jnp.pad(cb2, (0, _IB - _NCLS)).reshape(1, -1)
    mu, std, lg = _graph_stage(x, raw, glw_pad, W1, b1.reshape(1, -1), W2,
                               b2.reshape(1, -1), eps_noise, cW1,
                               cb1.reshape(1, -1), cw2_pad, cb2_pad)
    return (mu, std, lg[:, :_NCLS])


# speedup vs baseline: 6.8838x; 1.0705x over previous
import functools

import jax
import jax.numpy as jnp
import numpy as np
from jax import lax
from jax.experimental import pallas as pl
from jax.experimental.pallas import tpu as pltpu
from jax.experimental.pallas import tpu_sc as plsc

_G, _N, _D = 32, 256, 128
_P = 4
_HID = 256
_IB = 128
_NCLS = 10
_EPG = 4096
_EPSILON = 0.3
_SKIP = 0.2
_LANES = 16


def _sc_adjacency_body(ei, out, src_v, dst_v, adj_v, sem):
    nc = 2
    wid = lax.axis_index("s") * nc + lax.axis_index("c")
    cp_src = pltpu.async_copy(ei.at[0, wid], src_v, sem)
    cp_dst = pltpu.async_copy(ei.at[1, wid], dst_v, sem)

    zeros = jnp.zeros((_LANES,), jnp.float32)

    def zero_row(i, c):
        for j in range(_N // _LANES):
            adj_v[i, pl.ds(j * _LANES, _LANES)] = zeros
        return c

    lax.fori_loop(0, _N, zero_row, 0)
    cp_src.wait()
    cp_dst.wait()

    ones = jnp.ones((_LANES,), jnp.float32)

    def scatter_step(e, c):
        base = e * _LANES
        sv = src_v[pl.ds(base, _LANES)]
        dv = dst_v[pl.ds(base, _LANES)]
        plsc.store_scatter(adj_v, [sv, dv], ones)
        return c

    lax.fori_loop(0, _EPG // _LANES, scatter_step, 0)
    pltpu.sync_copy(adj_v, out.at[wid])


@functools.partial(
    pl.kernel,
    mesh=plsc.VectorSubcoreMesh(core_axis_name="c", subcore_axis_name="s"),
    out_type=jax.ShapeDtypeStruct((_G, _N, _N), jnp.float32),
    compiler_params=pltpu.CompilerParams(needs_layout_passes=False),
    scratch_types=[
        pltpu.VMEM((_EPG,), jnp.int32),
        pltpu.VMEM((_EPG,), jnp.int32),
        pltpu.VMEM((_N, _N), jnp.float32),
        pltpu.SemaphoreType.DMA,
    ],
)
def _sc_adjacency(ei, out, src_v, dst_v, adj_v, sem):
    _sc_adjacency_body(ei, out, src_v, dst_v, adj_v, sem)


_GPB = 4


def _graph_body(xg_ref, raw_ref, glw_ref, w1_ref, b1_ref, w2_ref, b2_ref,
                eps_ref, cw1_ref, cb1_ref, cw2_ref, cb2_ref,
                mu_ref, std_ref, lg_ref, embs_scr):
    lo = lax.Precision.DEFAULT
    g = pl.program_id(0)
    thresh = _P * float(np.float32(_EPSILON))

    for k in range(_GPB):
        xg = xg_ref[pl.ds(k * _N, _N), :]
        raw = raw_ref[k]

        parts = []
        for p in range(_P):
            ex = xg * glw_ref[p:p + 1, :]
            s = jnp.sum(ex * ex, axis=1, keepdims=True)
            parts.append(ex * lax.rsqrt(s))
        a_feat = jnp.concatenate(parts, axis=1)
        att = lax.dot_general(a_feat, a_feat, (((1,), (1,)), ((), ())),
                              precision=lo,
                              preferred_element_type=jnp.float32)

        bin_adj = jnp.where((raw > 0.0) | (att > thresh), 1.0, 0.0)
        deg = jnp.sum(bin_adj, axis=1, keepdims=True) + 1.0
        dinv = lax.rsqrt(deg)

        xw1 = jnp.dot(xg, w1_ref[...], precision=lo)
        dxw1 = dinv * xw1
        t1 = jnp.dot(bin_adj, dxw1, precision=lo) + dxw1
        h = jnp.maximum(dinv * t1 + b1_ref[...], 0.0)
        hw2 = jnp.dot(h, w2_ref[...], precision=lo)
        dhw2 = dinv * hw2
        o = (dinv * (jnp.dot(bin_adj, dhw2, precision=lo) + dhw2)
             + b2_ref[...])
        embs_scr[pl.ds(g * _GPB + k, 1), :] = jnp.mean(o, axis=0,
                                                       keepdims=True)

    @pl.when(g == _G // _GPB - 1)
    def _head():
        embs = embs_scr[...]
        mu = embs[:, :_IB]
        t = embs[:, _IB:] - float(_IB)
        std = jnp.maximum(t, 0.0) + jnp.log1p(jnp.exp(-jnp.abs(t)))
        z = mu + eps_ref[...] * std
        hc = jnp.maximum(
            jnp.dot(z, cw1_ref[...], precision=lo) + cb1_ref[...], 0.0)
        lg_ref[...] = jnp.dot(hc, cw2_ref[...], precision=lo) + cb2_ref[...]
        mu_ref[...] = mu
        std_ref[...] = std


def _graph_stage(x, raw, glw_pad, w1, b1_2d, w2, b2_2d,
                 eps, cw1, cb1_2d, cw2_pad, cb2_pad):
    zero2 = lambda g: (0, 0)
    return pl.pallas_call(
        _graph_body,
        grid=(_G // _GPB,),
        in_specs=[
            pl.BlockSpec((_GPB * _N, _D), lambda g: (g, 0)),
            pl.BlockSpec((_GPB, _N, _N), lambda g: (g, 0, 0)),
            pl.BlockSpec((8, _D), zero2),
            pl.BlockSpec((_D, _HID), zero2),
            pl.BlockSpec((1, _HID), zero2),
            pl.BlockSpec((_HID, 2 * _IB), zero2),
            pl.BlockSpec((1, 2 * _IB), zero2),
            pl.BlockSpec((_G, _IB), zero2),
            pl.BlockSpec((_IB, _IB), zero2),
            pl.BlockSpec((1, _IB), zero2),
            pl.BlockSpec((_IB, _IB), zero2),
            pl.BlockSpec((1, _IB), zero2),
        ],
        out_specs=(
            pl.BlockSpec((_G, _IB), zero2),
            pl.BlockSpec((_G, _IB), zero2),
            pl.BlockSpec((_G, _IB), zero2),
        ),
        out_shape=(
            jax.ShapeDtypeStruct((_G, _IB), jnp.float32),
            jax.ShapeDtypeStruct((_G, _IB), jnp.float32),
            jax.ShapeDtypeStruct((_G, _IB), jnp.float32),
        ),
        scratch_shapes=[pltpu.VMEM((_G, 2 * _IB), jnp.float32)],
    )(x, raw, glw_pad, w1, b1_2d, w2, b2_2d, eps, cw1, cb1_2d, cw2_pad,
      cb2_pad)


def kernel(x, edge_index, eps_noise, gl_weight, W1, b1, W2, b2,
           cW1, cb1, cW2, cb2):
    ei = edge_index.astype(jnp.int32)
    raw = _sc_adjacency(ei)

    glw_pad = jnp.pad(gl_weight, ((0, 8 - _P), (0, 0)))
    cw2_pad = jnp.pad(cW2, ((0, 0), (0, _IB - _NCLS)))
    cb2_pad = jnp.pad(cb2, (0, _IB - _NCLS)).reshape(1, -1)
    mu, std, lg = _graph_stage(x, raw, glw_pad, W1, b1.reshape(1, -1), W2,
                               b2.reshape(1, -1), eps_noise, cW1,
                               cb1.reshape(1, -1), cw2_pad, cb2_pad)
    return (mu, std, lg[:, :_NCLS])


# speedup vs baseline: 7.0990x; 1.0313x over previous
import functools

import jax
import jax.numpy as jnp
import numpy as np
from jax import lax
from jax.experimental import pallas as pl
from jax.experimental.pallas import tpu as pltpu
from jax.experimental.pallas import tpu_sc as plsc

_G, _N, _D = 32, 256, 128
_P = 4
_HID = 256
_IB = 128
_NCLS = 10
_EPG = 4096
_EPSILON = 0.3
_SKIP = 0.2
_LANES = 16


def _sc_adjacency_body(ei, out, src_v, dst_v, adj_v, sem):
    nc = 2
    wid = lax.axis_index("s") * nc + lax.axis_index("c")
    cp_src = pltpu.async_copy(ei.at[0, wid], src_v, sem)
    cp_dst = pltpu.async_copy(ei.at[1, wid], dst_v, sem)

    zeros = jnp.zeros((_LANES,), jnp.float32)

    def zero_row(i, c):
        for j in range(_N // _LANES):
            adj_v[i, pl.ds(j * _LANES, _LANES)] = zeros
        return c

    lax.fori_loop(0, _N, zero_row, 0)
    cp_src.wait()
    cp_dst.wait()

    ones = jnp.ones((_LANES,), jnp.float32)

    def scatter_step(e, c):
        base = e * _LANES
        sv = src_v[pl.ds(base, _LANES)]
        dv = dst_v[pl.ds(base, _LANES)]
        plsc.store_scatter(adj_v, [sv, dv], ones)
        return c

    lax.fori_loop(0, _EPG // _LANES, scatter_step, 0)
    pltpu.sync_copy(adj_v, out.at[wid])


@functools.partial(
    pl.kernel,
    mesh=plsc.VectorSubcoreMesh(core_axis_name="c", subcore_axis_name="s"),
    out_type=jax.ShapeDtypeStruct((_G, _N, _N), jnp.float32),
    compiler_params=pltpu.CompilerParams(needs_layout_passes=False),
    scratch_types=[
        pltpu.VMEM((_EPG,), jnp.int32),
        pltpu.VMEM((_EPG,), jnp.int32),
        pltpu.VMEM((_N, _N), jnp.float32),
        pltpu.SemaphoreType.DMA,
    ],
)
def _sc_adjacency(ei, out, src_v, dst_v, adj_v, sem):
    _sc_adjacency_body(ei, out, src_v, dst_v, adj_v, sem)


_GPB = 8


def _graph_body(xg_ref, raw_ref, glw_ref, w1_ref, b1_ref, w2_ref, b2_ref,
                eps_ref, cw1_ref, cb1_ref, cw2_ref, cb2_ref,
                mu_ref, std_ref, lg_ref, embs_scr):
    lo = lax.Precision.DEFAULT
    g = pl.program_id(0)
    thresh = _P * float(np.float32(_EPSILON))

    for k in range(_GPB):
        xg = xg_ref[pl.ds(k * _N, _N), :]
        raw = raw_ref[k]

        parts = []
        for p in range(_P):
            ex = xg * glw_ref[p:p + 1, :]
            s = jnp.sum(ex * ex, axis=1, keepdims=True)
            parts.append(ex * lax.rsqrt(s))
        a_feat = jnp.concatenate(parts, axis=1)
        att = lax.dot_general(a_feat, a_feat, (((1,), (1,)), ((), ())),
                              precision=lo,
                              preferred_element_type=jnp.float32)

        bin_adj = jnp.where((raw > 0.0) | (att > thresh), 1.0, 0.0)
        deg = jnp.sum(bin_adj, axis=1, keepdims=True) + 1.0
        dinv = lax.rsqrt(deg)

        xw1 = jnp.dot(xg, w1_ref[...], precision=lo)
        dxw1 = dinv * xw1
        t1 = jnp.dot(bin_adj, dxw1, precision=lo) + dxw1
        h = jnp.maximum(dinv * t1 + b1_ref[...], 0.0)
        hw2 = jnp.dot(h, w2_ref[...], precision=lo)
        dhw2 = dinv * hw2
        o = (dinv * (jnp.dot(bin_adj, dhw2, precision=lo) + dhw2)
             + b2_ref[...])
        embs_scr[pl.ds(g * _GPB + k, 1), :] = jnp.mean(o, axis=0,
                                                       keepdims=True)

    @pl.when(g == _G // _GPB - 1)
    def _head():
        embs = embs_scr[...]
        mu = embs[:, :_IB]
        t = embs[:, _IB:] - float(_IB)
        std = jnp.maximum(t, 0.0) + jnp.log1p(jnp.exp(-jnp.abs(t)))
        z = mu + eps_ref[...] * std
        hc = jnp.maximum(
            jnp.dot(z, cw1_ref[...], precision=lo) + cb1_ref[...], 0.0)
        lg_ref[...] = jnp.dot(hc, cw2_ref[...], precision=lo) + cb2_ref[...]
        mu_ref[...] = mu
        std_ref[...] = std


def _graph_stage(x, raw, glw_pad, w1, b1_2d, w2, b2_2d,
                 eps, cw1, cb1_2d, cw2_pad, cb2_pad):
    zero2 = lambda g: (0, 0)
    return pl.pallas_call(
        _graph_body,
        grid=(_G // _GPB,),
        in_specs=[
            pl.BlockSpec((_GPB * _N, _D), lambda g: (g, 0)),
            pl.BlockSpec((_GPB, _N, _N), lambda g: (g, 0, 0)),
            pl.BlockSpec((8, _D), zero2),
            pl.BlockSpec((_D, _HID), zero2),
            pl.BlockSpec((1, _HID), zero2),
            pl.BlockSpec((_HID, 2 * _IB), zero2),
            pl.BlockSpec((1, 2 * _IB), zero2),
            pl.BlockSpec((_G, _IB), zero2),
            pl.BlockSpec((_IB, _IB), zero2),
            pl.BlockSpec((1, _IB), zero2),
            pl.BlockSpec((_IB, _IB), zero2),
            pl.BlockSpec((1, _IB), zero2),
        ],
        out_specs=(
            pl.BlockSpec((_G, _IB), zero2),
            pl.BlockSpec((_G, _IB), zero2),
            pl.BlockSpec((_G, _IB), zero2),
        ),
        out_shape=(
            jax.ShapeDtypeStruct((_G, _IB), jnp.float32),
            jax.ShapeDtypeStruct((_G, _IB), jnp.float32),
            jax.ShapeDtypeStruct((_G, _IB), jnp.float32),
        ),
        scratch_shapes=[pltpu.VMEM((_G, 2 * _IB), jnp.float32)],
    )(x, raw, glw_pad, w1, b1_2d, w2, b2_2d, eps, cw1, cb1_2d, cw2_pad,
      cb2_pad)


def kernel(x, edge_index, eps_noise, gl_weight, W1, b1, W2, b2,
           cW1, cb1, cW2, cb2):
    ei = edge_index.astype(jnp.int32)
    raw = _sc_adjacency(ei)

    glw_pad = jnp.pad(gl_weight, ((0, 8 - _P), (0, 0)))
    cw2_pad = jnp.pad(cW2, ((0, 0), (0, _IB - _NCLS)))
    cb2_pad = jnp.pad(cb2, (0, _IB - _NCLS)).reshape(1, -1)
    mu, std, lg = _graph_stage(x, raw, glw_pad, W1, b1.reshape(1, -1), W2,
                               b2.reshape(1, -1), eps_noise, cW1,
                               cb1.reshape(1, -1), cw2_pad, cb2_pad)
    return (mu, std, lg[:, :_NCLS])
